# Initial kernel scaffold; baseline (speedup 1.0000x reference)
import jax
import jax.numpy as jnp
from jax.experimental import pallas as pl


def kernel(x, root_logits, rule_logits, emit_logits):
    raise NotImplementedError("write your pallas kernel here")


# speedup vs baseline: 14.7433x; 14.7433x over previous
import jax
import jax.numpy as jnp
from jax.experimental import pallas as pl
from jax.experimental.pallas import tpu as pltpu

_NT = 32
_T = 32
_B = 8
_V = 10000


def _inside_kernel(x_ref, root_ref, rule_ref, emit_ref, out_ref,
                   es_ref, er_ref, ms_ref, mr_ref):
    B, T, NT, V = _B, _T, _NT, _V

    emit = emit_ref[...]
    em = jnp.max(emit, axis=1, keepdims=True)
    es = jnp.sum(jnp.exp(emit - em), axis=1, keepdims=True)
    emit_n = emit - (em + jnp.log(es))

    x = x_ref[...]
    ids = jax.lax.broadcasted_iota(jnp.int32, (B * T, V), 1)
    oh = (ids == x).astype(jnp.float32)
    beta1 = jax.lax.dot_general(oh, emit_n, (((1,), (1,)), ((), ())),
                                preferred_element_type=jnp.float32)
    beta1 = beta1.reshape(B, T, NT)
    m1 = jnp.max(beta1, axis=-1)
    e1 = jnp.exp(beta1 - m1[..., None])

    rl = rule_ref[...]
    rm = jnp.max(rl, axis=1, keepdims=True)
    re = jnp.exp(rl - rm)
    rprob = re / jnp.sum(re, axis=1, keepdims=True)

    es_ref[1] = e1
    ms_ref[1] = m1
    er_ref[T] = e1
    mr_ref[T] = m1

    for w in range(2, T + 1):
        n = T - w + 1
        lo = T + 1 - w
        ls = es_ref[1:w, :, 0:n, :]
        rs = er_ref[lo + 1:T + 1, :, w - 1:T, :]
        sk = ms_ref[1:w, :, 0:n] + mr_ref[lo + 1:T + 1, :, w - 1:T]
        s = jnp.max(sk, axis=0)
        lw = ls * jnp.exp(sk - s[None])[..., None]
        prod = lw[:, :, :, :, None] * rs[:, :, :, None, :]
        c = jnp.sum(prod, axis=0)
        cf = c.reshape(B * n, NT * NT)
        v = jax.lax.dot_general(cf, rprob, (((1,), (1,)), ((), ())),
                                preferred_element_type=jnp.float32)
        vmax = jnp.max(v, axis=-1, keepdims=True)
        ew = (v / vmax).reshape(B, n, NT)
        mw = s + jnp.log(vmax).reshape(B, n)
        es_ref[w, :, 0:n, :] = ew
        ms_ref[w, :, 0:n] = mw
        er_ref[lo, :, w - 1:T, :] = ew
        mr_ref[lo, :, w - 1:T] = mw

    root = root_ref[...]
    rt = jnp.exp(root - jnp.max(root, axis=1, keepdims=True))
    rsm = rt / jnp.sum(rt, axis=1, keepdims=True)
    et = es_ref[T, :, 0, :]
    acc = jnp.sum(et * rsm, axis=1, keepdims=True)
    out_ref[...] = ms_ref[T, :, 0:1] + jnp.log(acc)


def kernel(x, root_logits, rule_logits, emit_logits):
    x2 = x.astype(jnp.int32).reshape(_B * _T, 1)
    root2 = root_logits.reshape(1, _NT)
    rule2 = rule_logits.reshape(_NT, _NT * _NT)
    ll = pl.pallas_call(
        _inside_kernel,
        out_shape=jax.ShapeDtypeStruct((_B, 1), jnp.float32),
        scratch_shapes=[
            pltpu.VMEM((_T + 1, _B, _T, _NT), jnp.float32),
            pltpu.VMEM((_T + 1, _B, _T, _NT), jnp.float32),
            pltpu.VMEM((_T + 1, _B, _T), jnp.float32),
            pltpu.VMEM((_T + 1, _B, _T), jnp.float32),
        ],
    )(x2, root2, rule2, emit_logits)
    return ll.reshape(_B)


# speedup vs baseline: 46.9694x; 3.1858x over previous
import jax
import jax.numpy as jnp
from jax.experimental import pallas as pl
from jax.experimental.pallas import tpu as pltpu

_NT = 32
_T = 32
_B = 8
_V = 10000


def _inside_kernel(x_ref, root_ref, rule_ref, emit_ref, out_ref,
                   es_ref, er_ref, ms_ref, mr_ref):
    B, T, NT, V = _B, _T, _NT, _V
    NN = NT * NT

    emit = emit_ref[...]
    em = jnp.max(emit, axis=1, keepdims=True)
    esum = jnp.sum(jnp.exp(emit - em), axis=1, keepdims=True)
    emit_n = emit - (em + jnp.log(esum))

    x = x_ref[...]
    ids = jax.lax.broadcasted_iota(jnp.int32, (T * B, V), 1)
    oh = (ids == x).astype(jnp.float32)
    beta1 = jax.lax.dot_general(oh, emit_n, (((1,), (1,)), ((), ())),
                                preferred_element_type=jnp.float32)
    m1 = jnp.max(beta1, axis=-1, keepdims=True)
    e1 = jnp.exp(beta1 - m1)

    rl = rule_ref[...]
    rm = jnp.max(rl, axis=1, keepdims=True)
    re = jnp.exp(rl - rm)
    rprob = re / jnp.sum(re, axis=1, keepdims=True)

    jj = jax.lax.broadcasted_iota(jnp.int32, (NT, NN), 1)
    row = jax.lax.broadcasted_iota(jnp.int32, (NT, NN), 0)
    tilemat = (jj % NT == row).astype(jnp.float32)
    repmat = (jj // NT == row).astype(jnp.float32)

    es_ref[1] = e1
    ms_ref[1] = m1
    er_ref[T] = e1
    mr_ref[T] = m1

    root = root_ref[...]
    rt = jnp.exp(root - jnp.max(root, axis=1, keepdims=True))
    rsm = rt / jnp.sum(rt, axis=1, keepdims=True)

    for w in range(2, T + 1):
        n8 = (T - w + 1) * B
        lo = T + 1 - w
        k = w - 1
        ls = es_ref[1:w, 0:n8, :]
        rs = er_ref[lo + 1:T + 1, (w - 1) * B:T * B, :]
        sk = ms_ref[1:w, 0:n8, :] + mr_ref[lo + 1:T + 1, (w - 1) * B:T * B, :]
        s = jnp.max(sk, axis=0)
        lw = (ls * jnp.exp(sk - s[None])).reshape(k * n8, NT)
        lt = jnp.dot(lw, tilemat, preferred_element_type=jnp.float32)
        rr = jnp.dot(rs.reshape(k * n8, NT), repmat,
                     preferred_element_type=jnp.float32)
        c = jnp.sum((lt * rr).reshape(k, n8, NN), axis=0)
        v = jax.lax.dot_general(c, rprob, (((1,), (1,)), ((), ())),
                                preferred_element_type=jnp.float32)
        if w < T:
            vmax = jnp.max(v, axis=-1, keepdims=True)
            ew = v / vmax
            mw = s + jnp.log(vmax)
            es_ref[w, 0:n8, :] = ew
            ms_ref[w, 0:n8, :] = mw
            er_ref[lo, (w - 1) * B:T * B, :] = ew
            mr_ref[lo, (w - 1) * B:T * B, :] = mw
        else:
            acc = jnp.sum(v * rsm, axis=1, keepdims=True)
            out_ref[...] = s + jnp.log(acc)


def kernel(x, root_logits, rule_logits, emit_logits):
    xt = x.astype(jnp.int32).T.reshape(_T * _B, 1)
    root2 = root_logits.reshape(1, _NT)
    rule2 = rule_logits.transpose(0, 2, 1).reshape(_NT, _NT * _NT)
    ll = pl.pallas_call(
        _inside_kernel,
        out_shape=jax.ShapeDtypeStruct((_B, 1), jnp.float32),
        scratch_shapes=[
            pltpu.VMEM((_T + 1, _T * _B, _NT), jnp.float32),
            pltpu.VMEM((_T + 1, _T * _B, _NT), jnp.float32),
            pltpu.VMEM((_T + 1, _T * _B, 1), jnp.float32),
            pltpu.VMEM((_T + 1, _T * _B, 1), jnp.float32),
        ],
    )(xt, root2, rule2, emit_logits)
    return ll.reshape(_B)
